# baseline (speedup 1.0000x reference)
import functools

import jax
import jax.numpy as jnp
from jax import lax
from jax.experimental import pallas as pl
from jax.experimental.pallas import tpu as pltpu
from jax.experimental.pallas import tpu_sc as plsc
from jax.experimental.layout import Format, Layout, with_layout_constraint

VOCAB = 1000000
SEQ = 200
BATCH = 4096
DIM = 64

NC, NS = 2, 16
NW = NC * NS
SB = SEQ // 8
BT = BATCH // 128
NITEM = SB * BT // NW
LANES = 16
RPAD = 65

_mesh = plsc.VectorSubcoreMesh(core_axis_name="c", subcore_axis_name="s")


@functools.partial(
    pl.kernel,
    mesh=_mesh,
    compiler_params=pltpu.CompilerParams(
        use_tc_tiling_on_sc=False, needs_layout_passes=False
    ),
    out_type=jax.ShapeDtypeStruct((SEQ, 8, BT, 8, 128), jnp.float32),
    scratch_types=[
        pltpu.VMEM((SEQ, DIM), jnp.float32),
        pltpu.VMEM((3, 8, 128), jnp.int32),
        pltpu.VMEM((4 * 128, DIM), jnp.float32),
        pltpu.VMEM((128, RPAD), jnp.float32),
        pltpu.VMEM((8, 8, 128), jnp.float32),
        pltpu.SemaphoreType.DMA,
        pltpu.SemaphoreType.DMA,
        pltpu.SemaphoreType.DMA,
        pltpu.SemaphoreType.DMA,
    ],
)
def _embed(idx_hbm, table_hbm, pos_hbm, out_hbm,
           pos_v, idx_v, rows_v, pad_v, st_v, *sems):
    wid = lax.axis_index("s") * NC + lax.axis_index("c")
    item0 = wid * NITEM
    pltpu.sync_copy(pos_hbm, pos_v)
    iota16 = lax.iota(jnp.int32, LANES)
    rvecs = [iota16 + bg * LANES for bg in range(8)]

    def stage_item(it, slot):
        sb = it // BT
        bt = lax.rem(it, BT)
        pltpu.sync_copy(idx_hbm.at[sb, bt], idx_v.at[slot])

    def gather(ring, si):
        return pltpu.async_copy(
            table_hbm.at[idx_v.at[ring, si]],
            rows_v.at[pl.ds((si % 4) * 128, 128)],
            sems[si % 4],
        )

    stage_item(item0, 0)
    for si in range(4):
        gather(0, si)

    def item_body(i, carry):
        it = item0 + i
        sb = it // BT
        bt = lax.rem(it, BT)
        iring = lax.rem(i, 3)
        nring = lax.rem(i + 1, 3)

        @pl.when(i + 1 < NITEM)
        def _stage_next():
            stage_item(it + 1, nring)

        for si in range(8):
            slot = si % 4
            pltpu.make_async_copy(
                table_hbm.at[idx_v.at[iring, si]],
                rows_v.at[pl.ds(slot * 128, 128)],
                sems[slot],
            ).wait()

            s = sb * 8 + si
            pvecs = [pos_v[s, pl.ds(cg * LANES, LANES)] for cg in range(4)]

            def pad_body(t, c2, _slot=slot, _pv=pvecs):
                for u in range(4):
                    b = t * 4 + u
                    r = _slot * 128 + b
                    for cg in range(4):
                        sl = pl.ds(cg * LANES, LANES)
                        pad_v[b, sl] = rows_v[r, sl] + _pv[cg]
                return c2

            lax.fori_loop(0, 32, pad_body, 0)

            if si < 4:
                gather(iring, si + 4)
            else:
                @pl.when(i + 1 < NITEM)
                def _refill(_si=si):
                    gather(nring, _si - 4)

            def dt_body(dt, c2):
                for dp in range(8):
                    c = dt * 8 + dp
                    cvec = jnp.full((LANES,), c, jnp.int32)
                    for bg in range(8):
                        vals = plsc.load_gather(pad_v, [rvecs[bg], cvec])
                        st_v[dt, dp, pl.ds(bg * LANES, LANES)] = vals
                return c2

            lax.fori_loop(0, 8, dt_body, 0)

            pltpu.sync_copy(st_v, out_hbm.at[s, :, bt])
        return carry

    lax.fori_loop(0, NITEM, item_body, 0)


def kernel(inputs, token_table, pos_table):
    idx4 = inputs.reshape(BT, 128, SB, 8).transpose(2, 0, 3, 1)
    out5 = _embed(idx4, token_table, pos_table)
    return out5.transpose(2, 4, 0, 1, 3).reshape(BATCH, SEQ, DIM)


# speedup vs baseline: 1.1198x; 1.1198x over previous
import functools

import jax
import jax.numpy as jnp
from jax import lax
from jax.experimental import pallas as pl
from jax.experimental.pallas import tpu as pltpu
from jax.experimental.pallas import tpu_sc as plsc

VOCAB = 1000000
SEQ = 200
BATCH = 4096
DIM = 64

NC, NS = 2, 16
NW = NC * NS
SB = SEQ // 8
BT = BATCH // 128
NITEM = SB * BT // NW
LANES = 16
STP = 129

_mesh = plsc.VectorSubcoreMesh(core_axis_name="c", subcore_axis_name="s")


@functools.partial(
    pl.kernel,
    mesh=_mesh,
    compiler_params=pltpu.CompilerParams(
        use_tc_tiling_on_sc=False, needs_layout_passes=False
    ),
    out_type=jax.ShapeDtypeStruct((SEQ, 8, BT, 8, 128), jnp.float32),
    scratch_types=[
        pltpu.VMEM((SEQ, DIM), jnp.float32),
        pltpu.VMEM((3, 8, 128), jnp.int32),
        pltpu.VMEM((8 * 128, DIM), jnp.float32),
        pltpu.VMEM((8, 8, STP), jnp.float32),
        pltpu.VMEM((8, 8, 128), jnp.float32),
        pltpu.SemaphoreType.DMA,
        pltpu.SemaphoreType.DMA,
        pltpu.SemaphoreType.DMA,
        pltpu.SemaphoreType.DMA,
        pltpu.SemaphoreType.DMA,
        pltpu.SemaphoreType.DMA,
        pltpu.SemaphoreType.DMA,
        pltpu.SemaphoreType.DMA,
    ],
)
def _embed(idx_hbm, table_hbm, pos_hbm, out_hbm,
           pos_v, idx_v, rows_v, st_v, cp_v, *sems):
    wid = lax.axis_index("s") * NC + lax.axis_index("c")
    item0 = wid * NITEM
    pltpu.sync_copy(pos_hbm, pos_v)
    iota16 = lax.iota(jnp.int32, LANES)
    dtv = [(iota16 + cg * LANES) // 8 for cg in range(4)]
    dpv = [lax.rem(iota16 + cg * LANES, 8) for cg in range(4)]

    def stage_idx(it, slot):
        sb = it // BT
        bt = lax.rem(it, BT)
        pltpu.sync_copy(idx_hbm.at[sb, bt], idx_v.at[slot])

    def issue_gather(islot, si):
        return pltpu.async_copy(
            table_hbm.at[idx_v.at[islot, si]],
            rows_v.at[pl.ds(si * 128, 128)],
            sems[si],
        )

    stage_idx(item0, 0)
    for si in range(8):
        issue_gather(0, si)

    def item_body(i, carry):
        it = item0 + i
        sb = it // BT
        bt = lax.rem(it, BT)
        islot = lax.rem(i, 3)
        nslot = lax.rem(i + 1, 3)

        @pl.when(i + 1 < NITEM)
        def _stage_next():
            stage_idx(it + 1, nslot)

        for si in range(8):
            pltpu.make_async_copy(
                table_hbm.at[idx_v.at[islot, si]],
                rows_v.at[pl.ds(si * 128, 128)],
                sems[si],
            ).wait()

            s = sb * 8 + si
            pvecs = [pos_v[s, pl.ds(cg * LANES, LANES)] for cg in range(4)]

            def b_body(t, c2, _si=si, _pv=pvecs):
                for u in range(4):
                    b = t * 4 + u
                    bv = jnp.full((LANES,), b, jnp.int32)
                    for cg in range(4):
                        vals = rows_v[_si * 128 + b, pl.ds(cg * LANES, LANES)]
                        plsc.store_scatter(
                            st_v, [dtv[cg], dpv[cg], bv], vals + _pv[cg]
                        )
                return c2

            lax.fori_loop(0, 32, b_body, 0)

            @pl.when(i + 1 < NITEM)
            def _refill(_si=si):
                issue_gather(nslot, _si)

            def cp_body(dt, c2):
                for dp in range(8):
                    for bg in range(8):
                        sl = pl.ds(bg * LANES, LANES)
                        cp_v[dt, dp, sl] = st_v[dt, dp, sl]
                return c2

            lax.fori_loop(0, 8, cp_body, 0)

            pltpu.sync_copy(cp_v, out_hbm.at[s, :, bt])
        return carry

    lax.fori_loop(0, NITEM, item_body, 0)


def kernel(inputs, token_table, pos_table):
    idx4 = inputs.reshape(BT, 128, SB, 8).transpose(2, 0, 3, 1)
    out5 = _embed(idx4, token_table, pos_table)
    return out5.transpose(2, 4, 0, 1, 3).reshape(BATCH, SEQ, DIM)


# speedup vs baseline: 1.5141x; 1.3521x over previous
import functools

import jax
import jax.numpy as jnp
from jax import lax
from jax.experimental import pallas as pl
from jax.experimental.pallas import tpu as pltpu
from jax.experimental.pallas import tpu_sc as plsc

VOCAB = 1000000
SEQ = 200
BATCH = 4096
DIM = 64

NC, NS = 2, 16
NW = NC * NS
SB = SEQ // 8
BT = BATCH // 128
NITEM = SB * BT // NW
LANES = 16
STP = 129

_mesh = plsc.VectorSubcoreMesh(core_axis_name="c", subcore_axis_name="s")


@functools.partial(
    pl.kernel,
    mesh=_mesh,
    compiler_params=pltpu.CompilerParams(
        use_tc_tiling_on_sc=False, needs_layout_passes=False
    ),
    out_type=jax.ShapeDtypeStruct((SEQ, 8, BT, 8, 128), jnp.float32),
    scratch_types=[
        pltpu.VMEM((SEQ, DIM), jnp.float32),
        pltpu.VMEM((3, 8, 128), jnp.int32),
        pltpu.VMEM((8 * 128, DIM), jnp.float32),
        pltpu.VMEM((8, 8, STP), jnp.float32),
        pltpu.VMEM((8, 8, STP), jnp.float32),
        pltpu.SemaphoreType.DMA,
        pltpu.SemaphoreType.DMA,
        pltpu.SemaphoreType.DMA,
        pltpu.SemaphoreType.DMA,
        pltpu.SemaphoreType.DMA,
        pltpu.SemaphoreType.DMA,
        pltpu.SemaphoreType.DMA,
        pltpu.SemaphoreType.DMA,
        pltpu.SemaphoreType.DMA,
    ],
)
def _embed(idx_hbm, table_hbm, pos_hbm, out_hbm,
           pos_v, idx_v, rows_v, st_a, st_b, osem, *sems):
    st_bufs = (st_a, st_b)
    wid = lax.axis_index("s") * NC + lax.axis_index("c")
    item0 = wid * NITEM
    pltpu.sync_copy(pos_hbm, pos_v)
    iota16 = lax.iota(jnp.int32, LANES)
    dtv = [(iota16 + cg * LANES) // 8 for cg in range(4)]
    dpv = [lax.rem(iota16 + cg * LANES, 8) for cg in range(4)]

    def stage_idx(it, slot):
        sb = it // BT
        bt = lax.rem(it, BT)
        pltpu.sync_copy(idx_hbm.at[sb, bt], idx_v.at[slot])

    def issue_gather(islot, si):
        return pltpu.async_copy(
            table_hbm.at[idx_v.at[islot, si]],
            rows_v.at[pl.ds(si * 128, 128)],
            sems[si],
        )

    stage_idx(item0, 0)
    for si in range(8):
        issue_gather(0, si)

    def item_body(i, carry):
        it = item0 + i
        sb = it // BT
        bt = lax.rem(it, BT)
        islot = lax.rem(i, 3)
        nslot = lax.rem(i + 1, 3)

        @pl.when(i + 1 < NITEM)
        def _stage_next():
            stage_idx(it + 1, nslot)

        for si in range(8):
            pltpu.make_async_copy(
                table_hbm.at[idx_v.at[islot, si]],
                rows_v.at[pl.ds(si * 128, 128)],
                sems[si],
            ).wait()

            s = sb * 8 + si
            st_v = st_bufs[si % 2]
            pvecs = [pos_v[s, pl.ds(cg * LANES, LANES)] for cg in range(4)]

            @pl.when(i * 8 + si >= 2)
            def _drain(_st=st_v, _s=s, _bt=bt):
                pltpu.make_async_copy(
                    _st.at[:, :, pl.ds(0, 128)],
                    out_hbm.at[_s, :, _bt],
                    osem,
                ).wait()

            def b_body(t, c2, _si=si, _pv=pvecs, _st=st_v):
                for u in range(8):
                    b = t * 8 + u
                    bv = jnp.full((LANES,), b, jnp.int32)
                    for cg in range(4):
                        vals = rows_v[_si * 128 + b, pl.ds(cg * LANES, LANES)]
                        plsc.store_scatter(
                            _st, [dtv[cg], dpv[cg], bv], vals + _pv[cg]
                        )
                return c2

            lax.fori_loop(0, 16, b_body, 0)

            @pl.when(i + 1 < NITEM)
            def _refill(_si=si):
                issue_gather(nslot, _si)

            pltpu.async_copy(
                st_v.at[:, :, pl.ds(0, 128)], out_hbm.at[s, :, bt], osem
            )
        return carry

    lax.fori_loop(0, NITEM, item_body, 0)

    for p in range(2):
        pltpu.make_async_copy(
            st_bufs[p].at[:, :, pl.ds(0, 128)],
            out_hbm.at[0, :, 0],
            osem,
        ).wait()


def kernel(inputs, token_table, pos_table):
    idx4 = inputs.reshape(BT, 128, SB, 8).transpose(2, 0, 3, 1)
    out5 = _embed(idx4, token_table, pos_table)
    return out5.transpose(2, 4, 0, 1, 3).reshape(BATCH, SEQ, DIM)
